# baseline (speedup 1.0000x reference)
import jax
import jax.numpy as jnp
from jax import lax
from jax.experimental import pallas as pl
from jax.experimental.pallas import tpu as pltpu
from jax.experimental.pallas import tpu_sc as plsc

B = 4096
H = 50
D = 64
NC = 2
NS = 16
NW = NC * NS
BW = B // NW
L = 16
NG = BW // L
NV = D // L
DP = 128

V = 100000
TC_FULL = V // DP
VP = (TC_FULL + 1) * DP
KMAX = 25
NB = 4
ROUNDS = (KMAX + NB - 1) // NB


def _transpose_col(stg_b, obuf_b, pitch, lane):
  def body(d, c):
    dv = jnp.full((L,), d, jnp.int32)
    for g in range(8):
      v = stg_b[d, pl.ds(g * L, L)]
      idx = (g * L + lane) * pitch + dv
      plsc.store_scatter(obuf_b, [idx], v)
    return c
  lax.fori_loop(0, D, body, 0)


def _relayout_body(ut_hbm, it_hbm, kt_hbm, utail, itail, ktail,
                   up_hbm, ip_hbm, kp_hbm,
                   stg0, stg1, stg2, stg3, ob0, ob1, ob2, ob3,
                   sem_in, sem_out):
  stg = [stg0, stg1, stg2, stg3]
  ob = [ob0, ob1, ob2, ob3]
  cid = lax.axis_index("c")
  sid = lax.axis_index("s")
  wid = sid * NC + cid
  lane = lax.iota(jnp.int32, L)

  zv = jnp.zeros((L,), jnp.float32)
  def zbody(i, c):
    for b in range(NB):
      ob[b][pl.ds(i * L, L)] = zv
    return c
  lax.fori_loop(0, DP * DP // L, zbody, 0)

  for tbl, tail, out, pitch, tail_wid in (
      (kt_hbm, ktail, kp_hbm, DP, 31),
      (ut_hbm, utail, up_hbm, D, 29),
      (it_hbm, itail, ip_hbm, D, 30),
  ):
    osz = DP * pitch

    def rbody(r, c, _tbl=tbl, _out=out, _pitch=pitch, _osz=osz):
      for b in range(NB):
        k = r * NB + b
        col = wid + NW * k
        ok = jnp.logical_and(k < KMAX, col < TC_FULL)

        @pl.when(ok)
        def _fire():
          pltpu.async_copy(
              _tbl.at[:, pl.ds(col * DP, DP)], stg[b], sem_in)

      for b in range(NB):
        k = r * NB + b
        col = wid + NW * k
        ok = jnp.logical_and(k < KMAX, col < TC_FULL)
        kp_ = k - NB
        colp = wid + NW * kp_
        okp = jnp.logical_and(kp_ >= 0, colp < TC_FULL)

        @pl.when(okp)
        def _drain_prev():
          pltpu.make_async_copy(
              ob[b].at[pl.ds(0, _osz)],
              _out.at[pl.ds(colp * _osz, _osz)], sem_out).wait()

        @pl.when(ok)
        def _work():
          pltpu.make_async_copy(
              _tbl.at[:, pl.ds(col * DP, DP)], stg[b], sem_in).wait()
          _transpose_col(stg[b], ob[b], _pitch, lane)
          pltpu.async_copy(
              ob[b].at[pl.ds(0, _osz)],
              _out.at[pl.ds(col * _osz, _osz)], sem_out)
      return c

    lax.fori_loop(0, ROUNDS, rbody, 0)

    for b in range(NB):
      k = (ROUNDS - 1) * NB + b
      col = wid + NW * k
      ok = jnp.logical_and(k < KMAX, col < TC_FULL)

      @pl.when(ok)
      def _drain_last(_out=out, _osz=osz, _col=col, _b=b):
        pltpu.make_async_copy(
            ob[_b].at[pl.ds(0, _osz)],
            _out.at[pl.ds(_col * _osz, _osz)], sem_out).wait()

    @pl.when(wid == tail_wid)
    def _tail(_tail=tail, _out=out, _pitch=pitch, _osz=osz):
      pltpu.sync_copy(_tail, stg[0])
      _transpose_col(stg[0], ob[0], _pitch, lane)
      pltpu.sync_copy(ob[0].at[pl.ds(0, _osz)],
                      _out.at[pl.ds(TC_FULL * _osz, _osz)])


_relayout = pl.kernel(
    _relayout_body,
    out_type=(
        jax.ShapeDtypeStruct((VP * D,), jnp.float32),
        jax.ShapeDtypeStruct((VP * D,), jnp.float32),
        jax.ShapeDtypeStruct((VP * DP,), jnp.float32),
    ),
    mesh=plsc.VectorSubcoreMesh(core_axis_name="c", subcore_axis_name="s"),
    scratch_types=(
        [pltpu.VMEM((D, DP), jnp.float32)] * NB +
        [pltpu.VMEM((DP * DP,), jnp.float32)] * NB +
        [pltpu.SemaphoreType.DMA, pltpu.SemaphoreType.DMA]),
    compiler_params=pltpu.CompilerParams(
        needs_layout_passes=False, use_tc_tiling_on_sc=True),
)


def _gather_body(user_hbm, item_hbm, kwt_hbm, up_hbm, ip_hbm, kp_hbm, out_hbm,
                 uidx, iidx, kidxt, upidx, ipidx, uh, ih,
                 urows, irows, acc, kw0, nzf, rcpf, outv, sem):
  cid = lax.axis_index("c")
  sid = lax.axis_index("s")
  wid = sid * NC + cid
  base = wid * BW

  pltpu.sync_copy(user_hbm.at[pl.ds(base, BW)], uidx)
  pltpu.sync_copy(item_hbm.at[pl.ds(base, BW)], iidx)
  pltpu.sync_copy(kwt_hbm.at[:, pl.ds(base, BW)], kidxt)

  for g in range(NG):
    sl = pl.ds(g * L, L)
    uv = uidx[sl]
    upidx[sl] = lax.shift_right_logical(uv, 1)
    uh[sl] = jnp.bitwise_and(uv, 1)
    iv = iidx[sl]
    ipidx[sl] = lax.shift_right_logical(iv, 1)
    ih[sl] = jnp.bitwise_and(iv, 1)

  cp_u = pltpu.async_copy(up_hbm.at[upidx], urows, sem)
  cp_i = pltpu.async_copy(ip_hbm.at[ipidx], irows, sem)
  cp_k0 = pltpu.async_copy(kp_hbm.at[kidxt.at[0]], acc, sem)
  pltpu.sync_copy(kp_hbm.at[0], kw0)

  for g in range(NG):
    def cnt_body(j, a, _g=g):
      ids = kidxt[j, pl.ds(_g * L, L)]
      return a + jnp.where(ids == 0, 1.0, 0.0)
    nz = lax.fori_loop(0, H, cnt_body, jnp.zeros((L,), jnp.float32))
    nzf[pl.ds(g * L, L)] = nz
    rcpf[pl.ds(g * L, L)] = 1.0 / jnp.maximum(jnp.float32(H) - nz, 1.0)

  cp_u.wait()
  cp_i.wait()
  cp_k0.wait()

  def fire(j, c):
    pltpu.async_copy(kp_hbm.at[kidxt.at[j]], acc, sem, add=True)
    return c
  lax.fori_loop(1, H, fire, 0)

  def drain(j, c):
    pltpu.make_async_copy(kp_hbm.at[kidxt.at[j]], acc, sem).wait()
    return c
  lax.fori_loop(1, H, drain, 0)

  lane = lax.iota(jnp.int32, L)
  last = lane == (L - 1)

  def fin(e, c):
    ev = jnp.full((L,), e, jnp.int32)
    nzv = plsc.load_gather(nzf, [ev])
    rcpv = plsc.load_gather(rcpf, [ev])
    uhm = plsc.load_gather(uh, [ev]) > 0
    ihm = plsc.load_gather(ih, [ev]) > 0
    s = jnp.zeros((L,), jnp.float32)
    for v in range(NV):
      sl = pl.ds(v * L, L)
      sh = pl.ds(D + v * L, L)
      uv = jnp.where(uhm, urows[e, sh], urows[e, sl])
      iv = jnp.where(ihm, irows[e, sh], irows[e, sl])
      ic = (acc[e, sl] - nzv * kw0[sl]) * rcpv
      s = s + uv * (iv + ic)
    cs = plsc.cumsum(s)
    plsc.store_scatter(outv, [ev], cs, mask=last)
    return c

  lax.fori_loop(0, BW, fin, 0)

  pltpu.sync_copy(outv, out_hbm.at[pl.ds(base, BW)])


_gather = pl.kernel(
    _gather_body,
    out_type=jax.ShapeDtypeStruct((B,), jnp.float32),
    mesh=plsc.VectorSubcoreMesh(core_axis_name="c", subcore_axis_name="s"),
    scratch_types=[
        pltpu.VMEM((BW,), jnp.int32),
        pltpu.VMEM((BW,), jnp.int32),
        pltpu.VMEM((H, BW), jnp.int32),
        pltpu.VMEM((BW,), jnp.int32),
        pltpu.VMEM((BW,), jnp.int32),
        pltpu.VMEM((BW,), jnp.int32),
        pltpu.VMEM((BW,), jnp.int32),
        pltpu.VMEM((BW, DP), jnp.float32),
        pltpu.VMEM((BW, DP), jnp.float32),
        pltpu.VMEM((BW, DP), jnp.float32),
        pltpu.VMEM((DP,), jnp.float32),
        pltpu.VMEM((BW,), jnp.float32),
        pltpu.VMEM((BW,), jnp.float32),
        pltpu.VMEM((BW,), jnp.float32),
        pltpu.SemaphoreType.DMA,
    ],
    compiler_params=pltpu.CompilerParams(
        needs_layout_passes=False, use_tc_tiling_on_sc=False),
)


def _tail_block(t):
  return jnp.pad(t[TC_FULL * DP:], ((0, VP - V), (0, 0))).T


@jax.jit
def kernel(user, item, keyword_ids, user_emb, item_id_emb, keyword_emb):
  kw_t = keyword_ids.astype(jnp.int32).T
  up1d, ip1d, kp1d = _relayout(
      user_emb.T, item_id_emb.T, keyword_emb.T,
      _tail_block(user_emb), _tail_block(item_id_emb),
      _tail_block(keyword_emb))
  return _gather(user.astype(jnp.int32), item.astype(jnp.int32), kw_t,
                 up1d.reshape(VP // 2, DP), ip1d.reshape(VP // 2, DP),
                 kp1d.reshape(VP, DP))
